# speedup vs baseline: 2.3743x; 2.3743x over previous
import functools

import jax
import jax.numpy as jnp
from jax import lax
from jax.experimental import pallas as pl
from jax.experimental.pallas import tpu as pltpu
from jax.experimental.pallas import tpu_sc as plsc

D_MODEL = 1024
N_IDX = 32768

_info = plsc.get_sparse_core_info()
_NC = _info.num_cores
_NS = _info.num_subcores
_NW = _NC * _NS
B_PER_W = N_IDX // _NW
CHUNK = 32
N_CHUNKS = B_PER_W // CHUNK

_mesh = plsc.VectorSubcoreMesh(core_axis_name="c", subcore_axis_name="s")


@functools.partial(
    pl.kernel,
    mesh=_mesh,
    out_type=jax.ShapeDtypeStruct((N_IDX, D_MODEL), jnp.float32),
    scratch_types=[
        pltpu.VMEM((B_PER_W,), jnp.int32),
        pltpu.VMEM((CHUNK, D_MODEL), jnp.float32),
        pltpu.VMEM((CHUNK, D_MODEL), jnp.float32),
        pltpu.SemaphoreType.DMA,
        pltpu.SemaphoreType.DMA,
    ],
)
def _gather_kernel(idx_hbm, pe_hbm, out_hbm, idx_v, buf0, buf1, sem0, sem1):
    wid = lax.axis_index("s") * _NC + lax.axis_index("c")
    base = wid * B_PER_W
    pltpu.sync_copy(idx_hbm.at[pl.ds(base, B_PER_W)], idx_v)

    bufs = (buf0, buf1)
    sems = (sem0, sem1)

    def start_gather(g, b):
        pltpu.async_copy(
            pe_hbm.at[idx_v.at[pl.ds(g * CHUNK, CHUNK)]], bufs[b], sems[b]
        )

    def wait_gather(b):
        pltpu.make_async_copy(
            pe_hbm.at[idx_v.at[pl.ds(0, CHUNK)]], bufs[b], sems[b]
        ).wait()

    start_gather(0, 0)

    @pl.loop(0, N_CHUNKS, step=2)
    def _(g0):
        for b in range(2):
            g = g0 + b

            @pl.when(g + 1 < N_CHUNKS)
            def _():
                start_gather(g + 1, 1 - b)

            wait_gather(b)
            pltpu.sync_copy(bufs[b], out_hbm.at[pl.ds(base + g * CHUNK, CHUNK)])


def kernel(x, pe):
    return _gather_kernel(x.reshape(-1), pe)
